# baseline (speedup 1.0000x reference)
import functools

import jax
import jax.numpy as jnp
from jax.experimental import pallas as pl
from jax.experimental.pallas import tpu as pltpu

TOP_K = 2
TM = 128
TN = 512
TB = 256


def _gating_body(x_ref, gwt_ref, gb_ref, gates_ref, idx_ref, *, n_experts):
    x = x_ref[...]
    logits = jnp.dot(x, gwt_ref[...], preferred_element_type=jnp.float32)
    logits = logits + gb_ref[...]
    m = jnp.max(logits, axis=1, keepdims=True)
    ex = jnp.exp(logits - m)
    p = ex / jnp.sum(ex, axis=1, keepdims=True)
    cols = jax.lax.broadcasted_iota(jnp.int32, p.shape, 1)
    v1 = jnp.max(p, axis=1, keepdims=True)
    i1 = jnp.min(jnp.where(p >= v1, cols, n_experts), axis=1, keepdims=True)
    pm = jnp.where(cols == i1, -1.0, p)
    v2 = jnp.max(pm, axis=1, keepdims=True)
    i2 = jnp.min(jnp.where(pm >= v2, cols, n_experts), axis=1, keepdims=True)
    gates_ref[...] = jnp.concatenate([v1, v2], axis=1)
    idx_ref[...] = jnp.concatenate([i1, i2], axis=1)


def _row_gather_body(srcs_ref, src_hbm, dst_hbm, sem, *, rows):
    def issue(r, c):
        pltpu.make_async_copy(
            src_hbm.at[pl.ds(srcs_ref[r], 1)],
            dst_hbm.at[pl.ds(r, 1)], sem).start()
        return c
    jax.lax.fori_loop(0, rows, issue, 0, unroll=8)

    def wait(r, c):
        pltpu.make_async_copy(
            src_hbm.at[pl.ds(0, 1)], dst_hbm.at[pl.ds(0, 1)], sem).wait()
        return c
    jax.lax.fori_loop(0, rows, wait, 0, unroll=8)


def _row_gather(srcs, src, rows, width, dtype):
    return pl.pallas_call(
        functools.partial(_row_gather_body, rows=rows),
        grid_spec=pltpu.PrefetchScalarGridSpec(
            num_scalar_prefetch=1,
            grid=(1,),
            in_specs=[pl.BlockSpec(memory_space=pl.ANY)],
            out_specs=pl.BlockSpec(memory_space=pl.ANY),
            scratch_shapes=[pltpu.SemaphoreType.DMA],
        ),
        out_shape=jax.ShapeDtypeStruct((rows, 8, width // 8), dtype),
    )(srcs, src.reshape(-1, 8, width // 8))


def _mm_body(tile_expert_ref, nact_ref, x_ref, w_ref, b_ref, out_ref):
    mt = pl.program_id(1)

    @pl.when(mt < nact_ref[0])
    def _compute():
        wb = w_ref[0].astype(jnp.bfloat16)
        acc = jnp.dot(x_ref[...], wb, preferred_element_type=jnp.float32)
        out_ref[...] = acc + b_ref[0]

    @pl.when(mt >= nact_ref[0])
    def _zero():
        out_ref[...] = jnp.zeros_like(out_ref)


def _combine_body(yg0_ref, yg1_ref, g0_ref, g1_ref, out_ref):
    out_ref[...] = (yg0_ref[0] * g0_ref[...] +
                    yg1_ref[0] * g1_ref[...]) * (1.0 / TOP_K)


def kernel(sequences, expert_weights, expert_biases, gating_w, gating_b):
    n, p, d = sequences.shape
    e, _, f = expert_weights.shape
    t = n * p
    s = t * TOP_K
    m_pad = s + e * TM
    m_tiles = m_pad // TM
    n_tiles = f // TN

    x = sequences.reshape(t, d)

    gates, idx = pl.pallas_call(
        functools.partial(_gating_body, n_experts=e),
        out_shape=[
            jax.ShapeDtypeStruct((t, TOP_K), jnp.float32),
            jax.ShapeDtypeStruct((t, TOP_K), jnp.int32),
        ],
    )(x, gating_w.T, gating_b.reshape(1, e))

    ei = idx.reshape(-1)
    onehot = (ei[:, None] == jnp.arange(e, dtype=ei.dtype)).astype(jnp.int32)
    incl = jnp.cumsum(onehot, axis=0)
    rank = jnp.take_along_axis(incl, ei[:, None], axis=1)[:, 0] - 1
    counts = incl[-1]
    padded = ((counts + TM - 1) // TM) * TM
    pad_end = jnp.cumsum(padded)
    pad_start = pad_end - padded
    dest = pad_start[ei] + rank
    nactive = (pad_end[-1] // TM).astype(jnp.int32).reshape(1)
    token_src = jnp.zeros((m_pad,), jnp.int32).at[dest].set(
        jnp.arange(s, dtype=jnp.int32) // TOP_K)
    tile_starts = jnp.arange(m_tiles, dtype=jnp.int32) * TM
    tile_expert = jnp.minimum(
        (tile_starts[:, None] >= pad_end[None, :]).sum(axis=1), e - 1
    ).astype(jnp.int32)

    xs = _row_gather(token_src, x, m_pad, d, jnp.float32)
    xs = xs.reshape(m_pad, d).astype(jnp.bfloat16)

    grid_spec = pltpu.PrefetchScalarGridSpec(
        num_scalar_prefetch=2,
        grid=(n_tiles, m_tiles),
        in_specs=[
            pl.BlockSpec((TM, d), lambda nt, mt, te, na: (mt, 0)),
            pl.BlockSpec((1, d, TN), lambda nt, mt, te, na: (te[mt], 0, nt)),
            pl.BlockSpec((1, 1, TN), lambda nt, mt, te, na: (te[mt], 0, nt)),
        ],
        out_specs=pl.BlockSpec((TM, TN), lambda nt, mt, te, na: (mt, nt)),
    )
    y = pl.pallas_call(
        _mm_body,
        grid_spec=grid_spec,
        out_shape=jax.ShapeDtypeStruct((m_pad, f), jnp.float32),
    )(tile_expert, nactive, xs, expert_weights,
      expert_biases.reshape(e, 1, f))

    comb_src = dest.reshape(t, TOP_K).T.reshape(-1).astype(jnp.int32)
    yg = _row_gather(comb_src, y, TOP_K * t, f, jnp.float32)
    yg = yg.reshape(TOP_K, t, 8, f // 8)

    g0 = gates[:, 0].reshape(t, 1, 1)
    g1 = gates[:, 1].reshape(t, 1, 1)
    out = pl.pallas_call(
        _combine_body,
        grid=(t // TB,),
        in_specs=[
            pl.BlockSpec((1, TB, 8, f // 8), lambda tb: (0, tb, 0, 0)),
            pl.BlockSpec((1, TB, 8, f // 8), lambda tb: (1, tb, 0, 0)),
            pl.BlockSpec((TB, 1, 1), lambda tb: (tb, 0, 0)),
            pl.BlockSpec((TB, 1, 1), lambda tb: (tb, 0, 0)),
        ],
        out_specs=pl.BlockSpec((TB, 8, f // 8), lambda tb: (tb, 0, 0)),
        out_shape=jax.ShapeDtypeStruct((t, 8, f // 8), jnp.float32),
    )(yg, yg, g0, g1)

    return out.reshape(n, p, f)


# speedup vs baseline: 5.9983x; 5.9983x over previous
import functools

import jax
import jax.numpy as jnp
from jax import lax
from jax.experimental import pallas as pl
from jax.experimental.pallas import tpu as pltpu
from jax.experimental.pallas import tpu_sc as plsc

TOP_K = 2
TM = 128
TN = 512
TB = 256

SC_NC = 2
SC_NS = 16
SC_NW = SC_NC * SC_NS


def _gating_body(x_ref, gwt_ref, gb_ref, gates_ref, idx_ref, *, n_experts):
    x = x_ref[...]
    logits = jnp.dot(x, gwt_ref[...], preferred_element_type=jnp.float32)
    logits = logits + gb_ref[...]
    m = jnp.max(logits, axis=1, keepdims=True)
    ex = jnp.exp(logits - m)
    p = ex / jnp.sum(ex, axis=1, keepdims=True)
    cols = jax.lax.broadcasted_iota(jnp.int32, p.shape, 1)
    v1 = jnp.max(p, axis=1, keepdims=True)
    i1 = jnp.min(jnp.where(p >= v1, cols, n_experts), axis=1, keepdims=True)
    pm = jnp.where(cols == i1, -1.0, p)
    v2 = jnp.max(pm, axis=1, keepdims=True)
    i2 = jnp.min(jnp.where(pm >= v2, cols, n_experts), axis=1, keepdims=True)
    gates_ref[...] = jnp.concatenate([v1, v2], axis=1)
    idx_ref[...] = jnp.concatenate([i1, i2], axis=1)


def _sc_gather(idx, table, chunk):
    rows = idx.shape[0]
    width = table.shape[1]
    dtype = table.dtype
    b_per_w = rows // SC_NW
    n_chunks = b_per_w // chunk
    mesh = plsc.VectorSubcoreMesh(core_axis_name="c", subcore_axis_name="s")

    @functools.partial(
        pl.kernel, mesh=mesh,
        out_type=jax.ShapeDtypeStruct((rows, width), dtype),
        scratch_types=[
            pltpu.VMEM((chunk,), jnp.int32),
            pltpu.VMEM((chunk, width), dtype),
            pltpu.SemaphoreType.DMA,
        ],
    )
    def k(idx_hbm, table_hbm, out_hbm, idx_v, rows_v, sem):
        wid = lax.axis_index("s") * SC_NC + lax.axis_index("c")
        base = wid * b_per_w

        def body(i, carry):
            off = base + i * chunk
            pltpu.sync_copy(idx_hbm.at[pl.ds(off, chunk)], idx_v)
            pltpu.async_copy(table_hbm.at[idx_v], rows_v, sem).wait()
            pltpu.sync_copy(rows_v, out_hbm.at[pl.ds(off, chunk)])
            return carry

        jax.lax.fori_loop(0, n_chunks, body, 0)

    return k(idx, table)


def _mm_body(tile_expert_ref, nact_ref, x_ref, w_ref, b_ref, out_ref):
    mt = pl.program_id(1)

    @pl.when(mt < nact_ref[0])
    def _compute():
        wb = w_ref[0].astype(jnp.bfloat16)
        acc = jnp.dot(x_ref[...], wb, preferred_element_type=jnp.float32)
        out_ref[...] = acc + b_ref[0]

    @pl.when(mt >= nact_ref[0])
    def _zero():
        out_ref[...] = jnp.zeros_like(out_ref)


def _combine_body(yg0_ref, yg1_ref, g0_ref, g1_ref, out_ref):
    out_ref[...] = (yg0_ref[0] * g0_ref[...] +
                    yg1_ref[0] * g1_ref[...]) * (1.0 / TOP_K)


def kernel(sequences, expert_weights, expert_biases, gating_w, gating_b):
    n, p, d = sequences.shape
    e, _, f = expert_weights.shape
    t = n * p
    s = t * TOP_K
    m_pad = s + e * TM
    m_tiles = m_pad // TM
    n_tiles = f // TN

    x = sequences.reshape(t, d)

    gates, idx = pl.pallas_call(
        functools.partial(_gating_body, n_experts=e),
        out_shape=[
            jax.ShapeDtypeStruct((t, TOP_K), jnp.float32),
            jax.ShapeDtypeStruct((t, TOP_K), jnp.int32),
        ],
    )(x, gating_w.T, gating_b.reshape(1, e))

    ei = idx.reshape(-1)
    onehot = (ei[:, None] == jnp.arange(e, dtype=ei.dtype)).astype(jnp.int32)
    incl = jnp.cumsum(onehot, axis=0)
    rank = jnp.take_along_axis(incl, ei[:, None], axis=1)[:, 0] - 1
    counts = incl[-1]
    padded = ((counts + TM - 1) // TM) * TM
    pad_end = jnp.cumsum(padded)
    pad_start = pad_end - padded
    dest = pad_start[ei] + rank
    nactive = (pad_end[-1] // TM).astype(jnp.int32).reshape(1)
    token_src = jnp.zeros((m_pad,), jnp.int32).at[dest].set(
        jnp.arange(s, dtype=jnp.int32) // TOP_K)
    tile_starts = jnp.arange(m_tiles, dtype=jnp.int32) * TM
    tile_expert = jnp.minimum(
        (tile_starts[:, None] >= pad_end[None, :]).sum(axis=1), e - 1
    ).astype(jnp.int32)

    xs = _sc_gather(token_src, x, 80).astype(jnp.bfloat16)

    grid_spec = pltpu.PrefetchScalarGridSpec(
        num_scalar_prefetch=2,
        grid=(n_tiles, m_tiles),
        in_specs=[
            pl.BlockSpec((TM, d), lambda nt, mt, te, na: (mt, 0)),
            pl.BlockSpec((1, d, TN), lambda nt, mt, te, na: (te[mt], 0, nt)),
            pl.BlockSpec((1, 1, TN), lambda nt, mt, te, na: (te[mt], 0, nt)),
        ],
        out_specs=pl.BlockSpec((TM, TN), lambda nt, mt, te, na: (mt, nt)),
    )
    y = pl.pallas_call(
        _mm_body,
        grid_spec=grid_spec,
        out_shape=jax.ShapeDtypeStruct((m_pad, f), jnp.float32),
    )(tile_expert, nactive, xs, expert_weights,
      expert_biases.reshape(e, 1, f))

    comb_src = dest.reshape(t, TOP_K).T.reshape(-1).astype(jnp.int32)
    yg = _sc_gather(comb_src, y, 16).reshape(TOP_K, t, f)

    g0 = gates[:, 0].reshape(t, 1)
    g1 = gates[:, 1].reshape(t, 1)
    out = pl.pallas_call(
        _combine_body,
        grid=(t // TB,),
        in_specs=[
            pl.BlockSpec((1, TB, f), lambda tb: (0, tb, 0)),
            pl.BlockSpec((1, TB, f), lambda tb: (1, tb, 0)),
            pl.BlockSpec((TB, 1), lambda tb: (tb, 0)),
            pl.BlockSpec((TB, 1), lambda tb: (tb, 0)),
        ],
        out_specs=pl.BlockSpec((TB, f), lambda tb: (tb, 0)),
        out_shape=jax.ShapeDtypeStruct((t, f), jnp.float32),
    )(yg, yg, g0, g1)

    return out.reshape(n, p, f)


# speedup vs baseline: 6.4052x; 1.0678x over previous
import functools

import jax
import jax.numpy as jnp
from jax import lax
from jax.experimental import pallas as pl
from jax.experimental.pallas import tpu as pltpu
from jax.experimental.pallas import tpu_sc as plsc

TOP_K = 2
TM = 128
TN = 512
TB = 256

SC_NC = 2
SC_NS = 16
SC_NW = SC_NC * SC_NS


def _gating_body(x_ref, gwt_ref, gb_ref, gates_ref, idx_ref, *, n_experts):
    x = x_ref[...]
    logits = jnp.dot(x, gwt_ref[...], preferred_element_type=jnp.float32)
    logits = logits + gb_ref[...]
    m = jnp.max(logits, axis=1, keepdims=True)
    ex = jnp.exp(logits - m)
    p = ex / jnp.sum(ex, axis=1, keepdims=True)
    cols = jax.lax.broadcasted_iota(jnp.int32, p.shape, 1)
    v1 = jnp.max(p, axis=1, keepdims=True)
    i1 = jnp.min(jnp.where(p >= v1, cols, n_experts), axis=1, keepdims=True)
    pm = jnp.where(cols == i1, -1.0, p)
    v2 = jnp.max(pm, axis=1, keepdims=True)
    i2 = jnp.min(jnp.where(pm >= v2, cols, n_experts), axis=1, keepdims=True)
    gates_ref[...] = jnp.concatenate([v1, v2], axis=1)
    idx_ref[...] = jnp.concatenate([i1, i2], axis=1)


def _sc_gather(idx, table, chunk):
    rows = idx.shape[0]
    width = table.shape[1]
    dtype = table.dtype
    b_per_w = rows // SC_NW
    n_chunks = b_per_w // chunk
    mesh = plsc.VectorSubcoreMesh(core_axis_name="c", subcore_axis_name="s")

    @functools.partial(
        pl.kernel, mesh=mesh,
        out_type=jax.ShapeDtypeStruct((rows, width), dtype),
        scratch_types=[
            pltpu.VMEM((chunk,), jnp.int32),
            pltpu.VMEM((chunk, width), dtype),
            pltpu.SemaphoreType.DMA,
        ],
    )
    def k(idx_hbm, table_hbm, out_hbm, idx_v, rows_v, sem):
        wid = lax.axis_index("s") * SC_NC + lax.axis_index("c")
        base = wid * b_per_w

        def body(i, carry):
            off = base + i * chunk
            pltpu.sync_copy(idx_hbm.at[pl.ds(off, chunk)], idx_v)
            pltpu.async_copy(table_hbm.at[idx_v], rows_v, sem).wait()
            pltpu.sync_copy(rows_v, out_hbm.at[pl.ds(off, chunk)])
            return carry

        jax.lax.fori_loop(0, n_chunks, body, 0)

    return k(idx, table)


def _mm_body(tile_expert_ref, nact_ref, x_hbm, w_ref, b_ref, out_ref,
             xs, sem, *, tm):
    nt = pl.program_id(0)
    mt = pl.program_id(1)

    @pl.when((nt == 0) & (mt == 0))
    def _load_x():
        pltpu.make_async_copy(x_hbm, xs, sem).start()
        pltpu.make_async_copy(x_hbm, xs, sem).wait()

    @pl.when(mt < nact_ref[0])
    def _compute():
        wb = w_ref[0].astype(jnp.bfloat16)
        acc = jnp.dot(xs[pl.ds(mt * tm, tm), :], wb,
                      preferred_element_type=jnp.float32)
        out_ref[...] = acc + b_ref[0]

    @pl.when(mt >= nact_ref[0])
    def _zero():
        out_ref[...] = jnp.zeros_like(out_ref)


def _combine_body(yg0_ref, yg1_ref, g0_ref, g1_ref, out_ref):
    out_ref[...] = (yg0_ref[0] * g0_ref[...] +
                    yg1_ref[0] * g1_ref[...]) * (1.0 / TOP_K)


def kernel(sequences, expert_weights, expert_biases, gating_w, gating_b):
    n, p, d = sequences.shape
    e, _, f = expert_weights.shape
    t = n * p
    s = t * TOP_K
    m_pad = s + e * TM
    m_tiles = m_pad // TM
    n_tiles = f // TN

    x = sequences.reshape(t, d)

    gates, idx = pl.pallas_call(
        functools.partial(_gating_body, n_experts=e),
        out_shape=[
            jax.ShapeDtypeStruct((t, TOP_K), jnp.float32),
            jax.ShapeDtypeStruct((t, TOP_K), jnp.int32),
        ],
    )(x, gating_w.T, gating_b.reshape(1, e))

    ei = idx.reshape(-1)
    onehot = (ei[:, None] == jnp.arange(e, dtype=ei.dtype)).astype(jnp.int32)
    incl = jnp.cumsum(onehot, axis=0)
    rank = jnp.take_along_axis(incl, ei[:, None], axis=1)[:, 0] - 1
    counts = incl[-1]
    padded = ((counts + TM - 1) // TM) * TM
    pad_end = jnp.cumsum(padded)
    pad_start = pad_end - padded
    dest = pad_start[ei] + rank
    nactive = (pad_end[-1] // TM).astype(jnp.int32).reshape(1)
    token_src = jnp.zeros((m_pad,), jnp.int32).at[dest].set(
        jnp.arange(s, dtype=jnp.int32) // TOP_K)
    tile_starts = jnp.arange(m_tiles, dtype=jnp.int32) * TM
    tile_expert = jnp.minimum(
        (tile_starts[:, None] >= pad_end[None, :]).sum(axis=1), e - 1
    ).astype(jnp.int32)

    xs = _sc_gather(token_src, x, 80).astype(jnp.bfloat16)

    grid_spec = pltpu.PrefetchScalarGridSpec(
        num_scalar_prefetch=2,
        grid=(n_tiles, m_tiles),
        in_specs=[
            pl.BlockSpec(memory_space=pl.ANY),
            pl.BlockSpec((1, d, TN), lambda nt, mt, te, na: (te[mt], 0, nt)),
            pl.BlockSpec((1, 1, TN), lambda nt, mt, te, na: (te[mt], 0, nt)),
        ],
        out_specs=pl.BlockSpec((TM, TN), lambda nt, mt, te, na: (mt, nt)),
        scratch_shapes=[
            pltpu.VMEM((m_pad, d), jnp.bfloat16),
            pltpu.SemaphoreType.DMA,
        ],
    )
    y = pl.pallas_call(
        functools.partial(_mm_body, tm=TM),
        grid_spec=grid_spec,
        out_shape=jax.ShapeDtypeStruct((m_pad, f), jnp.float32),
    )(tile_expert, nactive, xs, expert_weights,
      expert_biases.reshape(e, 1, f))

    comb_src = dest.reshape(t, TOP_K).T.reshape(-1).astype(jnp.int32)
    yg = _sc_gather(comb_src, y, 16).reshape(TOP_K, t, f)

    g0 = gates[:, 0].reshape(t, 1)
    g1 = gates[:, 1].reshape(t, 1)
    out = pl.pallas_call(
        _combine_body,
        grid=(t // TB,),
        in_specs=[
            pl.BlockSpec((1, TB, f), lambda tb: (0, tb, 0)),
            pl.BlockSpec((1, TB, f), lambda tb: (1, tb, 0)),
            pl.BlockSpec((TB, 1), lambda tb: (tb, 0)),
            pl.BlockSpec((TB, 1), lambda tb: (tb, 0)),
        ],
        out_specs=pl.BlockSpec((TB, f), lambda tb: (tb, 0)),
        out_shape=jax.ShapeDtypeStruct((t, f), jnp.float32),
    )(yg, yg, g0, g1)

    return out.reshape(n, p, f)


# speedup vs baseline: 6.9426x; 1.0839x over previous
import functools

import jax
import jax.numpy as jnp
from jax import lax
from jax.experimental import pallas as pl
from jax.experimental.pallas import tpu as pltpu
from jax.experimental.pallas import tpu_sc as plsc

TOP_K = 2
TM = 128
TN = 512

SC_NC = 2
SC_NS = 16
SC_NW = SC_NC * SC_NS


def _gating_body(x_ref, gwt_ref, gb_ref, gates_ref, idx_ref, *, n_experts):
    x = x_ref[...]
    logits = jnp.dot(x, gwt_ref[...], preferred_element_type=jnp.float32)
    logits = logits + gb_ref[...]
    m = jnp.max(logits, axis=1, keepdims=True)
    ex = jnp.exp(logits - m)
    p = ex / jnp.sum(ex, axis=1, keepdims=True)
    cols = jax.lax.broadcasted_iota(jnp.int32, p.shape, 1)
    v1 = jnp.max(p, axis=1, keepdims=True)
    i1 = jnp.min(jnp.where(p >= v1, cols, n_experts), axis=1, keepdims=True)
    pm = jnp.where(cols == i1, -1.0, p)
    v2 = jnp.max(pm, axis=1, keepdims=True)
    i2 = jnp.min(jnp.where(pm >= v2, cols, n_experts), axis=1, keepdims=True)
    gates_ref[...] = jnp.concatenate([v1, v2], axis=1)
    idx_ref[...] = jnp.concatenate([i1, i2], axis=1)


def _sc_distribute(dest_even, dest_odd, x, m_pad):
    t, width = x.shape
    tok_per_w = t // SC_NW
    mesh = plsc.VectorSubcoreMesh(core_axis_name="c", subcore_axis_name="s")

    @functools.partial(
        pl.kernel, mesh=mesh,
        out_type=jax.ShapeDtypeStruct((m_pad, width), x.dtype),
        scratch_types=[
            pltpu.VMEM((tok_per_w,), jnp.int32),
            pltpu.VMEM((tok_per_w,), jnp.int32),
            pltpu.VMEM((tok_per_w, width), x.dtype),
            pltpu.SemaphoreType.DMA,
        ],
    )
    def k(de_hbm, do_hbm, x_hbm, out_hbm, ie_v, io_v, rows_v, sem):
        wid = lax.axis_index("s") * SC_NC + lax.axis_index("c")
        base = wid * tok_per_w
        pltpu.sync_copy(de_hbm.at[pl.ds(base, tok_per_w)], ie_v)
        pltpu.sync_copy(do_hbm.at[pl.ds(base, tok_per_w)], io_v)
        pltpu.sync_copy(x_hbm.at[pl.ds(base, tok_per_w)], rows_v)
        pltpu.async_copy(rows_v, out_hbm.at[ie_v], sem).wait()
        pltpu.async_copy(rows_v, out_hbm.at[io_v], sem).wait()

    return k(dest_even, dest_odd, x)


def _sc_gather(idx, table, chunk):
    rows = idx.shape[0]
    width = table.shape[1]
    dtype = table.dtype
    b_per_w = rows // SC_NW
    n_chunks = b_per_w // chunk
    mesh = plsc.VectorSubcoreMesh(core_axis_name="c", subcore_axis_name="s")

    @functools.partial(
        pl.kernel, mesh=mesh,
        out_type=jax.ShapeDtypeStruct((rows, width), dtype),
        scratch_types=[
            pltpu.VMEM((chunk,), jnp.int32),
            pltpu.VMEM((chunk, width), dtype),
            pltpu.SemaphoreType.DMA,
        ],
    )
    def k(idx_hbm, table_hbm, out_hbm, idx_v, rows_v, sem):
        wid = lax.axis_index("s") * SC_NC + lax.axis_index("c")
        base = wid * b_per_w

        def body(i, carry):
            off = base + i * chunk
            pltpu.sync_copy(idx_hbm.at[pl.ds(off, chunk)], idx_v)
            pltpu.async_copy(table_hbm.at[idx_v], rows_v, sem).wait()
            pltpu.sync_copy(rows_v, out_hbm.at[pl.ds(off, chunk)])
            return carry

        jax.lax.fori_loop(0, n_chunks, body, 0)

    return k(idx, table)


def _combine_body(yg0_ref, yg1_ref, out_ref):
    out_ref[...] = yg0_ref[0] + yg1_ref[0]


def _mm_body(tile_expert_ref, nact_ref, x_hbm, w_ref, b_ref, gs_ref, out_ref,
             xs, wbf, sem, *, tm):
    nt = pl.program_id(0)
    mt = pl.program_id(1)

    @pl.when((nt == 0) & (mt == 0))
    def _load_x():
        pltpu.make_async_copy(x_hbm, xs, sem).start()
        pltpu.make_async_copy(x_hbm, xs, sem).wait()

    changed = (mt == 0) | (tile_expert_ref[mt] != tile_expert_ref[
        jnp.maximum(mt - 1, 0)])

    @pl.when(changed)
    def _cast_w():
        wbf[...] = w_ref[0].astype(jnp.bfloat16)

    @pl.when(mt < nact_ref[0])
    def _compute():
        acc = jnp.dot(xs[pl.ds(mt * tm, tm), :], wbf[...],
                      preferred_element_type=jnp.float32)
        out_ref[...] = (acc + b_ref[0]) * gs_ref[...]

    @pl.when(mt >= nact_ref[0])
    def _zero():
        out_ref[...] = jnp.zeros_like(out_ref)


def kernel(sequences, expert_weights, expert_biases, gating_w, gating_b):
    n, p, d = sequences.shape
    e, _, f = expert_weights.shape
    t = n * p
    s = t * TOP_K
    m_pad = s + e * TM
    m_tiles = m_pad // TM
    n_tiles = f // TN

    x = sequences.reshape(t, d)

    gates, idx = pl.pallas_call(
        functools.partial(_gating_body, n_experts=e),
        out_shape=[
            jax.ShapeDtypeStruct((t, TOP_K), jnp.float32),
            jax.ShapeDtypeStruct((t, TOP_K), jnp.int32),
        ],
    )(x, gating_w.T, gating_b.reshape(1, e))

    ei = idx.reshape(-1)
    onehot = (ei[:, None] == jnp.arange(e, dtype=ei.dtype)).astype(jnp.int32)
    incl = jnp.cumsum(onehot, axis=0)
    rank = jnp.take_along_axis(incl, ei[:, None], axis=1)[:, 0] - 1
    counts = incl[-1]
    padded = ((counts + TM - 1) // TM) * TM
    pad_end = jnp.cumsum(padded)
    pad_start = pad_end - padded
    dest = pad_start[ei] + rank
    nactive = (pad_end[-1] // TM).astype(jnp.int32).reshape(1)
    tile_starts = jnp.arange(m_tiles, dtype=jnp.int32) * TM
    tile_expert = jnp.minimum(
        (tile_starts[:, None] >= pad_end[None, :]).sum(axis=1), e - 1
    ).astype(jnp.int32)
    dest2 = dest.reshape(t, TOP_K)
    dest_even = dest2[:, 0].astype(jnp.int32)
    dest_odd = dest2[:, 1].astype(jnp.int32)
    gs = jnp.zeros((m_pad,), jnp.float32).at[dest].set(
        gates.reshape(-1) * (1.0 / TOP_K))

    xs = _sc_distribute(dest_even, dest_odd, x, m_pad).astype(jnp.bfloat16)

    grid_spec = pltpu.PrefetchScalarGridSpec(
        num_scalar_prefetch=2,
        grid=(n_tiles, m_tiles),
        in_specs=[
            pl.BlockSpec(memory_space=pl.ANY),
            pl.BlockSpec((1, d, TN), lambda nt, mt, te, na: (te[mt], 0, nt)),
            pl.BlockSpec((1, 1, TN), lambda nt, mt, te, na: (te[mt], 0, nt)),
            pl.BlockSpec((TM, 1), lambda nt, mt, te, na: (mt, 0)),
        ],
        out_specs=pl.BlockSpec((TM, TN), lambda nt, mt, te, na: (mt, nt)),
        scratch_shapes=[
            pltpu.VMEM((m_pad, d), jnp.bfloat16),
            pltpu.VMEM((d, TN), jnp.bfloat16),
            pltpu.SemaphoreType.DMA,
        ],
    )
    y = pl.pallas_call(
        functools.partial(_mm_body, tm=TM),
        grid_spec=grid_spec,
        out_shape=jax.ShapeDtypeStruct((m_pad, f), jnp.float32),
    )(tile_expert, nactive, xs, expert_weights,
      expert_biases.reshape(e, 1, f), gs.reshape(m_pad, 1))

    comb_src = jnp.concatenate([dest_even, dest_odd])
    yg = _sc_gather(comb_src, y, 16).reshape(TOP_K, t, f)

    tb = 256
    out = pl.pallas_call(
        _combine_body,
        grid=(t // tb,),
        in_specs=[
            pl.BlockSpec((1, tb, f), lambda i: (0, i, 0)),
            pl.BlockSpec((1, tb, f), lambda i: (1, i, 0)),
        ],
        out_specs=pl.BlockSpec((tb, f), lambda i: (i, 0)),
        out_shape=jax.ShapeDtypeStruct((t, f), jnp.float32),
    )(yg, yg)

    return out.reshape(n, p, f)


# speedup vs baseline: 7.0688x; 1.0182x over previous
import functools

import jax
import jax.numpy as jnp
from jax import lax
from jax.experimental import pallas as pl
from jax.experimental.pallas import tpu as pltpu
from jax.experimental.pallas import tpu_sc as plsc

TOP_K = 2
TM = 128
TN = 512

SC_NC = 2
SC_NS = 16
SC_NW = SC_NC * SC_NS


def _gating_body(x_ref, gwt_ref, gb_ref, gates_ref, idx_ref, *, n_experts):
    x = x_ref[...]
    logits = jnp.dot(x, gwt_ref[...], preferred_element_type=jnp.float32)
    logits = logits + gb_ref[...]
    m = jnp.max(logits, axis=1, keepdims=True)
    ex = jnp.exp(logits - m)
    p = ex / jnp.sum(ex, axis=1, keepdims=True)
    cols = jax.lax.broadcasted_iota(jnp.int32, p.shape, 1)
    v1 = jnp.max(p, axis=1, keepdims=True)
    i1 = jnp.min(jnp.where(p >= v1, cols, n_experts), axis=1, keepdims=True)
    pm = jnp.where(cols == i1, -1.0, p)
    v2 = jnp.max(pm, axis=1, keepdims=True)
    i2 = jnp.min(jnp.where(pm >= v2, cols, n_experts), axis=1, keepdims=True)
    gates_ref[...] = jnp.concatenate([v1, v2], axis=1)
    idx_ref[...] = jnp.concatenate([i1, i2], axis=1)


def _sc_distribute(dest_even, dest_odd, x, ge, go, m_pad):
    t, width = x.shape
    gw = ge.shape[1]
    tok_per_w = t // SC_NW
    mesh = plsc.VectorSubcoreMesh(core_axis_name="c", subcore_axis_name="s")

    @functools.partial(
        pl.kernel, mesh=mesh,
        out_type=[
            jax.ShapeDtypeStruct((m_pad, width), x.dtype),
            jax.ShapeDtypeStruct((m_pad, gw), jnp.float32),
        ],
        scratch_types=[
            pltpu.VMEM((tok_per_w,), jnp.int32),
            pltpu.VMEM((tok_per_w,), jnp.int32),
            pltpu.VMEM((tok_per_w, width), x.dtype),
            pltpu.VMEM((tok_per_w, gw), jnp.float32),
            pltpu.SemaphoreType.DMA,
        ],
    )
    def k(de_hbm, do_hbm, x_hbm, ge_hbm, go_hbm, out_hbm, gs_hbm,
          ie_v, io_v, rows_v, g_v, sem):
        wid = lax.axis_index("s") * SC_NC + lax.axis_index("c")
        base = wid * tok_per_w
        pltpu.sync_copy(de_hbm.at[pl.ds(base, tok_per_w)], ie_v)
        pltpu.sync_copy(do_hbm.at[pl.ds(base, tok_per_w)], io_v)
        pltpu.sync_copy(x_hbm.at[pl.ds(base, tok_per_w)], rows_v)
        pltpu.async_copy(rows_v, out_hbm.at[ie_v], sem).wait()
        pltpu.async_copy(rows_v, out_hbm.at[io_v], sem).wait()
        pltpu.sync_copy(ge_hbm.at[pl.ds(base, tok_per_w)], g_v)
        pltpu.async_copy(g_v, gs_hbm.at[ie_v], sem).wait()
        pltpu.sync_copy(go_hbm.at[pl.ds(base, tok_per_w)], g_v)
        pltpu.async_copy(g_v, gs_hbm.at[io_v], sem).wait()

    return k(dest_even, dest_odd, x, ge, go)


def _sc_gather(idx, table, chunk):
    rows = idx.shape[0]
    width = table.shape[1]
    dtype = table.dtype
    b_per_w = rows // SC_NW
    n_chunks = b_per_w // chunk
    mesh = plsc.VectorSubcoreMesh(core_axis_name="c", subcore_axis_name="s")

    @functools.partial(
        pl.kernel, mesh=mesh,
        out_type=jax.ShapeDtypeStruct((rows, width), dtype),
        scratch_types=[
            pltpu.VMEM((chunk,), jnp.int32),
            pltpu.VMEM((chunk, width), dtype),
            pltpu.SemaphoreType.DMA,
        ],
    )
    def k(idx_hbm, table_hbm, out_hbm, idx_v, rows_v, sem):
        wid = lax.axis_index("s") * SC_NC + lax.axis_index("c")
        base = wid * b_per_w

        def body(i, carry):
            off = base + i * chunk
            pltpu.sync_copy(idx_hbm.at[pl.ds(off, chunk)], idx_v)
            pltpu.async_copy(table_hbm.at[idx_v], rows_v, sem).wait()
            pltpu.sync_copy(rows_v, out_hbm.at[pl.ds(off, chunk)])
            return carry

        jax.lax.fori_loop(0, n_chunks, body, 0)

    return k(idx, table)


def _combine_body(yg0_ref, yg1_ref, out_ref):
    out_ref[...] = yg0_ref[0] + yg1_ref[0]


def _mm_body(tile_expert_ref, nact_ref, x_hbm, w_ref, b_ref, gs_ref, out_ref,
             xs, wbf, sem, *, tm):
    nt = pl.program_id(0)
    mt = pl.program_id(1)

    @pl.when((nt == 0) & (mt == 0))
    def _load_x():
        pltpu.make_async_copy(x_hbm, xs, sem).start()
        pltpu.make_async_copy(x_hbm, xs, sem).wait()

    changed = (mt == 0) | (tile_expert_ref[mt] != tile_expert_ref[
        jnp.maximum(mt - 1, 0)])

    @pl.when(changed)
    def _cast_w():
        wbf[...] = w_ref[0].astype(jnp.bfloat16)

    @pl.when(mt < nact_ref[0])
    def _compute():
        acc = jnp.dot(xs[pl.ds(mt * tm, tm), :], wbf[...],
                      preferred_element_type=jnp.float32)
        out_ref[...] = (acc + b_ref[0]) * gs_ref[:, 0:1]

    @pl.when(mt >= nact_ref[0])
    def _zero():
        out_ref[...] = jnp.zeros_like(out_ref)


def kernel(sequences, expert_weights, expert_biases, gating_w, gating_b):
    n, p, d = sequences.shape
    e, _, f = expert_weights.shape
    t = n * p
    s = t * TOP_K
    m_pad = s + e * TM
    m_tiles = m_pad // TM
    n_tiles = f // TN

    x = sequences.reshape(t, d)

    gates, idx = pl.pallas_call(
        functools.partial(_gating_body, n_experts=e),
        out_shape=[
            jax.ShapeDtypeStruct((t, TOP_K), jnp.float32),
            jax.ShapeDtypeStruct((t, TOP_K), jnp.int32),
        ],
    )(x, gating_w.T, gating_b.reshape(1, e))

    ei = idx.reshape(-1)
    onehot = (ei[:, None] == jnp.arange(e, dtype=ei.dtype)).astype(jnp.int32)
    incl = jnp.cumsum(onehot, axis=0)
    rank = jnp.take_along_axis(incl, ei[:, None], axis=1)[:, 0] - 1
    counts = incl[-1]
    padded = ((counts + TM - 1) // TM) * TM
    pad_end = jnp.cumsum(padded)
    pad_start = pad_end - padded
    dest = pad_start[ei] + rank
    nactive = (pad_end[-1] // TM).astype(jnp.int32).reshape(1)
    tile_starts = jnp.arange(m_tiles, dtype=jnp.int32) * TM
    tile_expert = jnp.minimum(
        (tile_starts[:, None] >= pad_end[None, :]).sum(axis=1), e - 1
    ).astype(jnp.int32)
    dest2 = dest.reshape(t, TOP_K)
    dest_even = dest2[:, 0].astype(jnp.int32)
    dest_odd = dest2[:, 1].astype(jnp.int32)
    ge = jnp.broadcast_to((gates[:, 0] * (1.0 / TOP_K))[:, None], (t, 128))
    go = jnp.broadcast_to((gates[:, 1] * (1.0 / TOP_K))[:, None], (t, 128))

    xs, gs = _sc_distribute(dest_even, dest_odd, x, ge, go, m_pad)
    xs = xs.astype(jnp.bfloat16)

    grid_spec = pltpu.PrefetchScalarGridSpec(
        num_scalar_prefetch=2,
        grid=(n_tiles, m_tiles),
        in_specs=[
            pl.BlockSpec(memory_space=pl.ANY),
            pl.BlockSpec((1, d, TN), lambda nt, mt, te, na: (te[mt], 0, nt)),
            pl.BlockSpec((1, 1, TN), lambda nt, mt, te, na: (te[mt], 0, nt)),
            pl.BlockSpec((TM, 128), lambda nt, mt, te, na: (mt, 0)),
        ],
        out_specs=pl.BlockSpec((TM, TN), lambda nt, mt, te, na: (mt, nt)),
        scratch_shapes=[
            pltpu.VMEM((m_pad, d), jnp.bfloat16),
            pltpu.VMEM((d, TN), jnp.bfloat16),
            pltpu.SemaphoreType.DMA,
        ],
    )
    y = pl.pallas_call(
        functools.partial(_mm_body, tm=TM),
        grid_spec=grid_spec,
        out_shape=jax.ShapeDtypeStruct((m_pad, f), jnp.float32),
    )(tile_expert, nactive, xs, expert_weights,
      expert_biases.reshape(e, 1, f), gs)

    comb_src = jnp.concatenate([dest_even, dest_odd])
    yg = _sc_gather(comb_src, y, 16).reshape(TOP_K, t, f)

    tb = 256
    out = pl.pallas_call(
        _combine_body,
        grid=(t // tb,),
        in_specs=[
            pl.BlockSpec((1, tb, f), lambda i: (0, i, 0)),
            pl.BlockSpec((1, tb, f), lambda i: (1, i, 0)),
        ],
        out_specs=pl.BlockSpec((tb, f), lambda i: (i, 0)),
        out_shape=jax.ShapeDtypeStruct((t, f), jnp.float32),
    )(yg, yg)

    return out.reshape(n, p, f)
